# Initial kernel scaffold; baseline (speedup 1.0000x reference)
import functools

import jax
import jax.numpy as jnp
from jax import lax
from jax.experimental import pallas as pl
from jax.experimental.pallas import tpu as pltpu
from jax.experimental.pallas import tpu_sc as plsc

N = 10000
E = 320000
D = 128

NC = 2
NS = 16
NW = NC * NS
CH = 80
CPT = E // (CH * NW)
NPAD = 10240
ZROWS = NPAD // NS
XROWS = N // NS

_mesh = plsc.VectorSubcoreMesh(core_axis_name="c", subcore_axis_name="s")


@functools.partial(
    pl.kernel,
    out_type=jax.ShapeDtypeStruct((NC, N, 16), jnp.float32),
    mesh=_mesh,
    scratch_types=[
        pltpu.VMEM((CPT, CH), jnp.int32),
        pltpu.VMEM((CH, 16), jnp.float32),
        pltpu.VMEM((ZROWS, 16), jnp.float32),
        pltpu.VMEM_SHARED((NPAD, 16), jnp.float32),
    ],
)
def _sc_degree(dst_hbm, out_hbm, idx_v, ones_v, zero_v, deg_sh):
    c = lax.axis_index("c")
    s = lax.axis_index("s")
    wid = s * NC + c

    def fill(i, _):
        ones_v[i, :] = jnp.ones((16,), jnp.float32)
        return 0

    lax.fori_loop(0, CH, fill, 0)

    def zfill(i, _):
        zero_v[i, :] = jnp.zeros((16,), jnp.float32)
        return 0

    lax.fori_loop(0, ZROWS, zfill, 0)
    pltpu.sync_copy(zero_v, deg_sh.at[pl.ds(s * ZROWS, ZROWS)])
    plsc.subcore_barrier()

    pltpu.sync_copy(dst_hbm.at[pl.ds(wid * CPT, CPT)], idx_v)

    def body(k, _):
        pltpu.sync_copy(ones_v, deg_sh.at[idx_v.at[k]], add=True)
        return 0

    lax.fori_loop(0, CPT, body, 0)
    plsc.subcore_barrier()
    pltpu.sync_copy(deg_sh.at[pl.ds(s * XROWS, XROWS)],
                    out_hbm.at[c, pl.ds(s * XROWS, XROWS)])


@functools.partial(
    pl.kernel,
    out_type=jax.ShapeDtypeStruct((NC, N, D), jnp.float32),
    mesh=_mesh,
    scratch_types=[
        pltpu.VMEM((CPT, CH), jnp.int32),
        pltpu.VMEM((CPT, CH), jnp.int32),
        pltpu.VMEM((2, CH, D), jnp.float32),
        pltpu.VMEM_SHARED((NPAD, D), jnp.float32),
        pltpu.SemaphoreType.DMA,
        pltpu.SemaphoreType.DMA,
    ],
)
def _sc_spmm(q_hbm, src_hbm, dst_hbm, out_hbm, src_v, dst_v, rows_v, s_sh,
             sem0, sem1):
    c = lax.axis_index("c")
    s = lax.axis_index("s")
    wid = s * NC + c

    def zrow(i, _):
        for j in range(D // 16):
            rows_v[0, i, pl.ds(j * 16, 16)] = jnp.zeros((16,), jnp.float32)
        return 0

    lax.fori_loop(0, CH, zrow, 0)
    for kk in range(ZROWS // CH):
        pltpu.sync_copy(rows_v.at[0], s_sh.at[pl.ds(s * ZROWS + kk * CH, CH)])
    plsc.subcore_barrier()

    pltpu.sync_copy(src_hbm.at[pl.ds(wid * CPT, CPT)], src_v)
    pltpu.sync_copy(dst_hbm.at[pl.ds(wid * CPT, CPT)], dst_v)

    pltpu.async_copy(q_hbm.at[src_v.at[0]], rows_v.at[0], sem0)

    def body(k, _):
        b = lax.rem(k, 2)

        @pl.when(k + 1 < CPT)
        def _():
            @pl.when(b == 0)
            def _():
                pltpu.async_copy(q_hbm.at[src_v.at[k + 1]], rows_v.at[1], sem1)

            @pl.when(b == 1)
            def _():
                pltpu.async_copy(q_hbm.at[src_v.at[k + 1]], rows_v.at[0], sem0)

        @pl.when(b == 0)
        def _():
            pltpu.make_async_copy(q_hbm.at[src_v.at[k]], rows_v.at[0],
                                  sem0).wait()
            pltpu.sync_copy(rows_v.at[0], s_sh.at[dst_v.at[k]], add=True)

        @pl.when(b == 1)
        def _():
            pltpu.make_async_copy(q_hbm.at[src_v.at[k]], rows_v.at[1],
                                  sem1).wait()
            pltpu.sync_copy(rows_v.at[1], s_sh.at[dst_v.at[k]], add=True)

        return 0

    lax.fori_loop(0, CPT, body, 0)
    plsc.subcore_barrier()
    pltpu.sync_copy(s_sh.at[pl.ds(s * XROWS, XROWS)],
                    out_hbm.at[c, pl.ds(s * XROWS, XROWS)])


RB = 1000


def _tc1_body(deg_ref, x_ref, w_ref, q_ref, dis_ref):
    deg = deg_ref[0] + deg_ref[1]
    dis = lax.rsqrt(deg + 1.0)
    pre = jnp.dot(x_ref[...], w_ref[...], preferred_element_type=jnp.float32)
    q_ref[...] = pre * dis[:, 0:1]
    dis_ref[...] = dis


def _tc2_body(sp_ref, q1_ref, dis_ref, w_ref, q2_ref):
    dis = dis_ref[:, 0:1]
    h1 = jnp.maximum((sp_ref[0] + sp_ref[1] + q1_ref[...]) * dis, 0.0)
    q2_ref[...] = jnp.dot(h1, w_ref[...],
                          preferred_element_type=jnp.float32) * dis


def _tc3_body(sp_ref, q2_ref, dis_ref, out_ref):
    dis = dis_ref[:, 0:1]
    out_ref[...] = jnp.maximum((sp_ref[0] + sp_ref[1] + q2_ref[...]) * dis,
                               0.0)


_tc1 = pl.pallas_call(
    _tc1_body,
    grid=(N // RB,),
    in_specs=[
        pl.BlockSpec((NC, RB, 16), lambda i: (0, i, 0)),
        pl.BlockSpec((RB, D), lambda i: (i, 0)),
        pl.BlockSpec((D, D), lambda i: (0, 0)),
    ],
    out_specs=[
        pl.BlockSpec((RB, D), lambda i: (i, 0)),
        pl.BlockSpec((RB, 16), lambda i: (i, 0)),
    ],
    out_shape=[
        jax.ShapeDtypeStruct((N, D), jnp.float32),
        jax.ShapeDtypeStruct((N, 16), jnp.float32),
    ],
)

_tc2 = pl.pallas_call(
    _tc2_body,
    grid=(N // RB,),
    in_specs=[
        pl.BlockSpec((NC, RB, D), lambda i: (0, i, 0)),
        pl.BlockSpec((RB, D), lambda i: (i, 0)),
        pl.BlockSpec((RB, 16), lambda i: (i, 0)),
        pl.BlockSpec((D, D), lambda i: (0, 0)),
    ],
    out_specs=pl.BlockSpec((RB, D), lambda i: (i, 0)),
    out_shape=jax.ShapeDtypeStruct((N, D), jnp.float32),
)

_tc3 = pl.pallas_call(
    _tc3_body,
    grid=(N // RB,),
    in_specs=[
        pl.BlockSpec((NC, RB, D), lambda i: (0, i, 0)),
        pl.BlockSpec((RB, D), lambda i: (i, 0)),
        pl.BlockSpec((RB, 16), lambda i: (i, 0)),
    ],
    out_specs=pl.BlockSpec((RB, D), lambda i: (i, 0)),
    out_shape=jax.ShapeDtypeStruct((N, D), jnp.float32),
)


def kernel(x, edge_index, W1, W2):
    src = edge_index[0].reshape(E // CH, CH)
    dst = edge_index[1].reshape(E // CH, CH)
    deg2 = _sc_degree(dst)
    q1, dis = _tc1(deg2, x, W1)
    s1 = _sc_spmm(q1, src, dst)
    q2 = _tc2(s1, q1, dis, W2)
    s2 = _sc_spmm(q2, src, dst)
    return _tc3(s2, q2, dis)


# speedup vs baseline: 13.9883x; 13.9883x over previous
import functools

import jax
import jax.numpy as jnp
from jax import lax
from jax.experimental import pallas as pl
from jax.experimental.pallas import tpu as pltpu
from jax.experimental.pallas import tpu_sc as plsc

N = 10000
E = 320000
D = 128
CW = 64

NC = 2
NS = 16
NW = NC * NS
CH = 80
CPT = E // (CH * NW)
NPAD = 10240
ZROWS = NPAD // NS

_mesh = plsc.VectorSubcoreMesh(core_axis_name="c", subcore_axis_name="s")


@functools.partial(
    pl.kernel,
    out_type=[
        jax.ShapeDtypeStruct((NC, NPAD, CW), jnp.float32),
        jax.ShapeDtypeStruct((NC, NPAD, CW), jnp.float32),
    ],
    mesh=_mesh,
    compiler_params=pltpu.CompilerParams(use_tc_tiling_on_sc=False),
    scratch_types=[
        pltpu.VMEM((CPT, CH), jnp.int32),
        pltpu.VMEM((CPT, CH), jnp.int32),
        pltpu.VMEM((2, CH, CW), jnp.float32),
        pltpu.VMEM_SHARED((NPAD, CW), jnp.float32),
        pltpu.SemaphoreType.DMA,
        pltpu.SemaphoreType.DMA,
    ],
)
def _sc_spmm(qlo_hbm, qhi_hbm, src_hbm, dst_hbm, outlo_hbm, outhi_hbm,
             src_v, dst_v, rows_v, s_sh, sem0, sem1):
    c = lax.axis_index("c")
    s = lax.axis_index("s")
    wid = s * NC + c

    pltpu.sync_copy(src_hbm.at[wid], src_v)
    pltpu.sync_copy(dst_hbm.at[wid], dst_v)

    for h in range(2):
        q_hbm = qlo_hbm if h == 0 else qhi_hbm
        out_hbm = outlo_hbm if h == 0 else outhi_hbm

        def zrow(i, _):
            for j in range(CW // 16):
                rows_v[0, i, pl.ds(j * 16, 16)] = jnp.zeros((16,),
                                                            jnp.float32)
            return 0

        lax.fori_loop(0, CH, zrow, 0)
        for kk in range(ZROWS // CH):
            pltpu.sync_copy(rows_v.at[0],
                            s_sh.at[pl.ds(s * ZROWS + kk * CH, CH)])
        plsc.subcore_barrier()

        pltpu.async_copy(q_hbm.at[src_v.at[0]], rows_v.at[0], sem0)

        def body(k, _):
            b = lax.rem(k, 2)

            @pl.when(k + 1 < CPT)
            def _():
                @pl.when(b == 0)
                def _():
                    pltpu.async_copy(q_hbm.at[src_v.at[k + 1]], rows_v.at[1],
                                     sem1)

                @pl.when(b == 1)
                def _():
                    pltpu.async_copy(q_hbm.at[src_v.at[k + 1]], rows_v.at[0],
                                     sem0)

            @pl.when(b == 0)
            def _():
                pltpu.make_async_copy(q_hbm.at[src_v.at[k]], rows_v.at[0],
                                      sem0).wait()
                pltpu.sync_copy(rows_v.at[0], s_sh.at[dst_v.at[k]], add=True)

            @pl.when(b == 1)
            def _():
                pltpu.make_async_copy(q_hbm.at[src_v.at[k]], rows_v.at[1],
                                      sem1).wait()
                pltpu.sync_copy(rows_v.at[1], s_sh.at[dst_v.at[k]], add=True)

            return 0

        lax.fori_loop(0, CPT, body, 0)
        plsc.subcore_barrier()
        pltpu.sync_copy(s_sh.at[pl.ds(s * ZROWS, ZROWS)],
                        out_hbm.at[c, pl.ds(s * ZROWS, ZROWS)])
        plsc.subcore_barrier()


RB = 1000


def _tc1_body(deg_ref, x_ref, w_ref, qlo_ref, qhi_ref, dis_ref):
    deg = (deg_ref[0] + deg_ref[1])[:, 0:16]
    dis = lax.rsqrt(deg + 1.0)
    pre = jnp.dot(x_ref[...], w_ref[...], preferred_element_type=jnp.float32)
    q = pre * dis[:, 0:1]
    qlo_ref[...] = q[:, 0:CW]
    qhi_ref[...] = q[:, CW:D]
    dis_ref[...] = dis


def _tc2_body(slo_ref, shi_ref, qlo_ref, qhi_ref, dis_ref, w_ref,
              q2lo_ref, q2hi_ref):
    dis = dis_ref[:, 0:1]
    hlo = jnp.maximum((slo_ref[0] + slo_ref[1] + qlo_ref[...]) * dis, 0.0)
    hhi = jnp.maximum((shi_ref[0] + shi_ref[1] + qhi_ref[...]) * dis, 0.0)
    pre = (jnp.dot(hlo, w_ref[0:CW, :], preferred_element_type=jnp.float32)
           + jnp.dot(hhi, w_ref[CW:D, :], preferred_element_type=jnp.float32))
    q2 = pre * dis
    q2lo_ref[...] = q2[:, 0:CW]
    q2hi_ref[...] = q2[:, CW:D]


def _tc3_body(slo_ref, shi_ref, qlo_ref, qhi_ref, dis_ref, out_ref):
    dis = dis_ref[:, 0:1]
    out_ref[:, 0:CW] = jnp.maximum(
        (slo_ref[0] + slo_ref[1] + qlo_ref[...]) * dis, 0.0)
    out_ref[:, CW:D] = jnp.maximum(
        (shi_ref[0] + shi_ref[1] + qhi_ref[...]) * dis, 0.0)


_spec_nc = pl.BlockSpec((NC, RB, CW), lambda i: (0, i, 0))
_spec_q = pl.BlockSpec((RB, CW), lambda i: (i, 0))
_spec_dis = pl.BlockSpec((RB, 16), lambda i: (i, 0))
_spec_row = pl.BlockSpec((RB, D), lambda i: (i, 0))
_spec_w = pl.BlockSpec((D, D), lambda i: (0, 0))

_tc1 = pl.pallas_call(
    _tc1_body,
    grid=(N // RB,),
    in_specs=[_spec_nc, _spec_row, _spec_w],
    out_specs=[_spec_q, _spec_q, _spec_dis],
    out_shape=[
        jax.ShapeDtypeStruct((N, CW), jnp.float32),
        jax.ShapeDtypeStruct((N, CW), jnp.float32),
        jax.ShapeDtypeStruct((N, 16), jnp.float32),
    ],
)

_tc2 = pl.pallas_call(
    _tc2_body,
    grid=(N // RB,),
    in_specs=[_spec_nc, _spec_nc, _spec_q, _spec_q, _spec_dis, _spec_w],
    out_specs=[_spec_q, _spec_q],
    out_shape=[
        jax.ShapeDtypeStruct((N, CW), jnp.float32),
        jax.ShapeDtypeStruct((N, CW), jnp.float32),
    ],
)

_tc3 = pl.pallas_call(
    _tc3_body,
    grid=(N // RB,),
    in_specs=[_spec_nc, _spec_nc, _spec_q, _spec_q, _spec_dis],
    out_specs=_spec_row,
    out_shape=jax.ShapeDtypeStruct((N, D), jnp.float32),
)


def kernel(x, edge_index, W1, W2):
    src = edge_index[0].reshape(NW, CPT, CH)
    dst = edge_index[1].reshape(NW, CPT, CH)
    ones = jnp.ones((N, CW), jnp.float32)
    deg2, _ = _sc_spmm(ones, ones, src, dst)
    q1lo, q1hi, dis = _tc1(deg2, x, W1)
    s1lo, s1hi = _sc_spmm(q1lo, q1hi, src, dst)
    q2lo, q2hi = _tc2(s1lo, s1hi, q1lo, q1hi, dis, W2)
    s2lo, s2hi = _sc_spmm(q2lo, q2hi, src, dst)
    return _tc3(s2lo, s2hi, q2lo, q2hi, dis)


# speedup vs baseline: 20.0894x; 1.4362x over previous
import functools

import jax
import jax.numpy as jnp
from jax import lax
from jax.experimental import pallas as pl
from jax.experimental.pallas import tpu as pltpu
from jax.experimental.pallas import tpu_sc as plsc

N = 10000
E = 320000
D = 128
CW = 64

NC = 2
NS = 16
NW = NC * NS
CH = 80
CPT = E // (CH * NW)
NPAD = 10240
ZROWS = NPAD // NS

_mesh = plsc.VectorSubcoreMesh(core_axis_name="c", subcore_axis_name="s")


@functools.partial(
    pl.kernel,
    out_type=jax.ShapeDtypeStruct((NC, NPAD, 16), jnp.float32),
    mesh=_mesh,
    compiler_params=pltpu.CompilerParams(use_tc_tiling_on_sc=False),
    scratch_types=[
        pltpu.VMEM((CPT, CH), jnp.int32),
        pltpu.VMEM((CH, 16), jnp.float32),
        pltpu.VMEM_SHARED((NPAD, 16), jnp.float32),
        pltpu.SemaphoreType.DMA,
    ],
)
def _sc_degree(dst_hbm, out_hbm, idx_v, ones_v, deg_sh, sem):
    c = lax.axis_index("c")
    s = lax.axis_index("s")
    wid = s * NC + c

    def zfill(i, _):
        ones_v[i, :] = jnp.zeros((16,), jnp.float32)
        return 0

    lax.fori_loop(0, CH, zfill, 0)
    for kk in range(ZROWS // CH):
        pltpu.sync_copy(ones_v, deg_sh.at[pl.ds(s * ZROWS + kk * CH, CH)])

    def ofill(i, _):
        ones_v[i, :] = jnp.ones((16,), jnp.float32)
        return 0

    lax.fori_loop(0, CH, ofill, 0)
    plsc.subcore_barrier()

    pltpu.sync_copy(dst_hbm.at[wid], idx_v)

    DEPTH = 8

    def body(k, _):
        pltpu.async_copy(ones_v, deg_sh.at[idx_v.at[k]], sem, add=True)

        @pl.when(k >= DEPTH)
        def _():
            pltpu.make_async_copy(ones_v, deg_sh.at[idx_v.at[k - DEPTH]],
                                  sem).wait()

        return 0

    lax.fori_loop(0, CPT, body, 0)
    for t in range(DEPTH):
        pltpu.make_async_copy(ones_v, deg_sh.at[idx_v.at[CPT - DEPTH + t]],
                              sem).wait()
    plsc.subcore_barrier()
    pltpu.sync_copy(deg_sh.at[pl.ds(s * ZROWS, ZROWS)],
                    out_hbm.at[c, pl.ds(s * ZROWS, ZROWS)])


@functools.partial(
    pl.kernel,
    out_type=[
        jax.ShapeDtypeStruct((NC, NPAD, CW), jnp.float32),
        jax.ShapeDtypeStruct((NC, NPAD, CW), jnp.float32),
    ],
    mesh=_mesh,
    compiler_params=pltpu.CompilerParams(use_tc_tiling_on_sc=False),
    scratch_types=[
        pltpu.VMEM((CPT, CH), jnp.int32),
        pltpu.VMEM((CPT, CH), jnp.int32),
        pltpu.VMEM((4, CH, CW), jnp.float32),
        pltpu.VMEM_SHARED((NPAD, CW), jnp.float32),
        pltpu.SemaphoreType.DMA,
        pltpu.SemaphoreType.DMA,
        pltpu.SemaphoreType.DMA,
        pltpu.SemaphoreType.DMA,
        pltpu.SemaphoreType.DMA,
        pltpu.SemaphoreType.DMA,
        pltpu.SemaphoreType.DMA,
        pltpu.SemaphoreType.DMA,
    ],
)
def _sc_spmm(qlo_hbm, qhi_hbm, src_hbm, dst_hbm, outlo_hbm, outhi_hbm,
             src_v, dst_v, rows_v, s_sh,
             g0, g1, g2, g3, s0, s1, s2, s3):
    c = lax.axis_index("c")
    s = lax.axis_index("s")
    wid = s * NC + c

    pltpu.sync_copy(src_hbm.at[wid], src_v)
    pltpu.sync_copy(dst_hbm.at[wid], dst_v)

    for h in range(2):
        q_hbm = qlo_hbm if h == 0 else qhi_hbm
        out_hbm = outlo_hbm if h == 0 else outhi_hbm

        def zrow(i, _):
            for j in range(CW // 16):
                rows_v[0, i, pl.ds(j * 16, 16)] = jnp.zeros((16,),
                                                            jnp.float32)
            return 0

        lax.fori_loop(0, CH, zrow, 0)
        for kk in range(ZROWS // CH):
            pltpu.sync_copy(rows_v.at[0],
                            s_sh.at[pl.ds(s * ZROWS + kk * CH, CH)])
        plsc.subcore_barrier()

        gsem = (g0, g1, g2, g3)
        ssem = (s0, s1, s2, s3)
        pltpu.async_copy(q_hbm.at[src_v.at[0]], rows_v.at[0], g0)
        pltpu.async_copy(q_hbm.at[src_v.at[1]], rows_v.at[1], g1)

        def body(k, _):
            b = lax.rem(k, 4)
            for i in range(4):
                j = (i + 2) % 4

                @pl.when(b == i)
                def _(i=i, j=j):
                    pltpu.make_async_copy(q_hbm.at[src_v.at[k]],
                                          rows_v.at[i], gsem[i]).wait()
                    pltpu.async_copy(rows_v.at[i], s_sh.at[dst_v.at[k]],
                                     ssem[i], add=True)

                    @pl.when(k >= 2)
                    def _():
                        pltpu.make_async_copy(rows_v.at[j],
                                              s_sh.at[dst_v.at[k - 2]],
                                              ssem[j]).wait()

                    @pl.when(k + 2 < CPT)
                    def _():
                        pltpu.async_copy(q_hbm.at[src_v.at[k + 2]],
                                         rows_v.at[j], gsem[j])

            return 0

        lax.fori_loop(0, CPT, body, 0)
        pltpu.make_async_copy(rows_v.at[(CPT - 2) % 4],
                              s_sh.at[dst_v.at[CPT - 2]],
                              ssem[(CPT - 2) % 4]).wait()
        pltpu.make_async_copy(rows_v.at[(CPT - 1) % 4],
                              s_sh.at[dst_v.at[CPT - 1]],
                              ssem[(CPT - 1) % 4]).wait()
        plsc.subcore_barrier()
        pltpu.sync_copy(s_sh.at[pl.ds(s * ZROWS, ZROWS)],
                        out_hbm.at[c, pl.ds(s * ZROWS, ZROWS)])
        plsc.subcore_barrier()


RB = 1000


def _tc1_body(deg_ref, x_ref, w_ref, qlo_ref, qhi_ref, dis_ref):
    deg = deg_ref[0] + deg_ref[1]
    dis = lax.rsqrt(deg + 1.0)
    pre = jnp.dot(x_ref[...], w_ref[...], preferred_element_type=jnp.float32)
    q = pre * dis[:, 0:1]
    qlo_ref[...] = q[:, 0:CW]
    qhi_ref[...] = q[:, CW:D]
    dis_ref[...] = dis


def _tc2_body(slo_ref, shi_ref, qlo_ref, qhi_ref, dis_ref, w_ref,
              q2lo_ref, q2hi_ref):
    dis = dis_ref[:, 0:1]
    hlo = jnp.maximum((slo_ref[0] + slo_ref[1] + qlo_ref[...]) * dis, 0.0)
    hhi = jnp.maximum((shi_ref[0] + shi_ref[1] + qhi_ref[...]) * dis, 0.0)
    pre = (jnp.dot(hlo, w_ref[0:CW, :], preferred_element_type=jnp.float32)
           + jnp.dot(hhi, w_ref[CW:D, :], preferred_element_type=jnp.float32))
    q2 = pre * dis
    q2lo_ref[...] = q2[:, 0:CW]
    q2hi_ref[...] = q2[:, CW:D]


def _tc3_body(slo_ref, shi_ref, qlo_ref, qhi_ref, dis_ref, out_ref):
    dis = dis_ref[:, 0:1]
    out_ref[:, 0:CW] = jnp.maximum(
        (slo_ref[0] + slo_ref[1] + qlo_ref[...]) * dis, 0.0)
    out_ref[:, CW:D] = jnp.maximum(
        (shi_ref[0] + shi_ref[1] + qhi_ref[...]) * dis, 0.0)


_spec_nc = pl.BlockSpec((NC, RB, CW), lambda i: (0, i, 0))
_spec_deg = pl.BlockSpec((NC, RB, 16), lambda i: (0, i, 0))
_spec_q = pl.BlockSpec((RB, CW), lambda i: (i, 0))
_spec_dis = pl.BlockSpec((RB, 16), lambda i: (i, 0))
_spec_row = pl.BlockSpec((RB, D), lambda i: (i, 0))
_spec_w = pl.BlockSpec((D, D), lambda i: (0, 0))

_tc1 = pl.pallas_call(
    _tc1_body,
    grid=(N // RB,),
    in_specs=[_spec_deg, _spec_row, _spec_w],
    out_specs=[_spec_q, _spec_q, _spec_dis],
    out_shape=[
        jax.ShapeDtypeStruct((N, CW), jnp.float32),
        jax.ShapeDtypeStruct((N, CW), jnp.float32),
        jax.ShapeDtypeStruct((N, 16), jnp.float32),
    ],
)

_tc2 = pl.pallas_call(
    _tc2_body,
    grid=(N // RB,),
    in_specs=[_spec_nc, _spec_nc, _spec_q, _spec_q, _spec_dis, _spec_w],
    out_specs=[_spec_q, _spec_q],
    out_shape=[
        jax.ShapeDtypeStruct((N, CW), jnp.float32),
        jax.ShapeDtypeStruct((N, CW), jnp.float32),
    ],
)

_tc3 = pl.pallas_call(
    _tc3_body,
    grid=(N // RB,),
    in_specs=[_spec_nc, _spec_nc, _spec_q, _spec_q, _spec_dis],
    out_specs=_spec_row,
    out_shape=jax.ShapeDtypeStruct((N, D), jnp.float32),
)


def kernel(x, edge_index, W1, W2):
    src = edge_index[0].reshape(NW, CPT, CH)
    dst = edge_index[1].reshape(NW, CPT, CH)
    deg2 = _sc_degree(dst)
    q1lo, q1hi, dis = _tc1(deg2, x, W1)
    s1lo, s1hi = _sc_spmm(q1lo, q1hi, src, dst)
    q2lo, q2hi = _tc2(s1lo, s1hi, q1lo, q1hi, dis, W2)
    s2lo, s2hi = _sc_spmm(q2lo, q2hi, src, dst)
    return _tc3(s2lo, s2hi, q2lo, q2hi, dis)


# speedup vs baseline: 25.9155x; 1.2900x over previous
import functools

import jax
import jax.numpy as jnp
from jax import lax
from jax.experimental import pallas as pl
from jax.experimental.pallas import tpu as pltpu
from jax.experimental.pallas import tpu_sc as plsc

N = 10000
E = 320000
D = 128
CW = 64

NC = 2
NS = 16
NW = NC * NS
CH = 80
EPT = -(-E // (NW * CH)) * CH
EPAD = EPT * NW
CPT = EPT // CH
NPAD = 10240
NBUF = 12
AHEAD = 8
ZROWS = NPAD // NS

_mesh = plsc.VectorSubcoreMesh(core_axis_name="c", subcore_axis_name="s")


@functools.partial(
    pl.kernel,
    out_type=jax.ShapeDtypeStruct((NC, NPAD, 16), jnp.float32),
    mesh=_mesh,
    compiler_params=pltpu.CompilerParams(use_tc_tiling_on_sc=False),
    scratch_types=[
        pltpu.VMEM((CPT, CH), jnp.int32),
        pltpu.VMEM((CH, 16), jnp.float32),
        pltpu.VMEM_SHARED((NPAD, 16), jnp.float32),
        pltpu.SemaphoreType.DMA,
    ],
)
def _sc_degree(edge_hbm, out_hbm, idx_v, ones_v, deg_sh, sem):
    c = lax.axis_index("c")
    s = lax.axis_index("s")
    wid = s * NC + c

    def zfill(i, _):
        ones_v[i, :] = jnp.zeros((16,), jnp.float32)
        return 0

    lax.fori_loop(0, CH, zfill, 0)
    for kk in range(ZROWS // CH):
        pltpu.sync_copy(ones_v, deg_sh.at[pl.ds(s * ZROWS + kk * CH, CH)])

    def ofill(i, _):
        ones_v[i, :] = jnp.ones((16,), jnp.float32)
        return 0

    lax.fori_loop(0, CH, ofill, 0)
    plsc.subcore_barrier()

    pltpu.sync_copy(edge_hbm.at[1, wid], idx_v)

    DEPTH = 8

    def body(k, _):
        pltpu.async_copy(ones_v, deg_sh.at[idx_v.at[k]], sem, add=True)

        @pl.when(k >= DEPTH)
        def _():
            pltpu.make_async_copy(ones_v, deg_sh.at[idx_v.at[k - DEPTH]],
                                  sem).wait()

        return 0

    lax.fori_loop(0, CPT, body, 0)
    for t in range(DEPTH):
        pltpu.make_async_copy(ones_v, deg_sh.at[idx_v.at[CPT - DEPTH + t]],
                              sem).wait()
    plsc.subcore_barrier()
    pltpu.sync_copy(deg_sh.at[pl.ds(s * ZROWS, ZROWS)],
                    out_hbm.at[c, pl.ds(s * ZROWS, ZROWS)])


@functools.partial(
    pl.kernel,
    out_type=jax.ShapeDtypeStruct((NC, NPAD, CW), jnp.float32),
    mesh=_mesh,
    compiler_params=pltpu.CompilerParams(use_tc_tiling_on_sc=False),
    scratch_types=[
        pltpu.VMEM((CPT, CH), jnp.int32),
        pltpu.VMEM((CPT, CH), jnp.int32),
        pltpu.VMEM((NBUF, CH, CW), jnp.float32),
        pltpu.VMEM_SHARED((NPAD, CW), jnp.float32),
    ] + [pltpu.SemaphoreType.DMA] * (2 * NBUF),
)
def _sc_spmm(q_hbm, edge_hbm, out_hbm,
             src_v, dst_v, rows_v, s_sh, *sems):
    c = lax.axis_index("c")
    s = lax.axis_index("s")
    wid = s * NC + c

    pltpu.sync_copy(edge_hbm.at[0, wid], src_v)
    pltpu.sync_copy(edge_hbm.at[1, wid], dst_v)

    def zrow(i, _):
        for j in range(CW // 16):
            rows_v[0, i, pl.ds(j * 16, 16)] = jnp.zeros((16,), jnp.float32)
        return 0

    lax.fori_loop(0, CH, zrow, 0)
    for kk in range(ZROWS // CH):
        pltpu.sync_copy(rows_v.at[0],
                        s_sh.at[pl.ds(s * ZROWS + kk * CH, CH)])
    plsc.subcore_barrier()

    gsem = sems[:NBUF]
    ssem = sems[NBUF:]
    for p in range(AHEAD):
        pltpu.async_copy(q_hbm.at[src_v.at[p]], rows_v.at[p], gsem[p])

    def body(k, _):
        b = lax.rem(k, NBUF)
        for i in range(NBUF):
            j = (i + AHEAD) % NBUF

            @pl.when(b == i)
            def _(i=i, j=j):
                pltpu.make_async_copy(q_hbm.at[src_v.at[k]],
                                      rows_v.at[i], gsem[i]).wait()
                pltpu.async_copy(rows_v.at[i], s_sh.at[dst_v.at[k]],
                                 ssem[i], add=True)

                @pl.when(k >= NBUF - AHEAD)
                def _():
                    pltpu.make_async_copy(rows_v.at[j],
                                          s_sh.at[dst_v.at[k - (NBUF - AHEAD)]],
                                          ssem[j]).wait()

                @pl.when(k + AHEAD < CPT)
                def _():
                    pltpu.async_copy(q_hbm.at[src_v.at[k + AHEAD]],
                                     rows_v.at[j], gsem[j])

        return 0

    lax.fori_loop(0, CPT, body, 0)
    for t in range(NBUF - AHEAD):
        kk2 = CPT - (NBUF - AHEAD) + t
        pltpu.make_async_copy(rows_v.at[kk2 % NBUF], s_sh.at[dst_v.at[kk2]],
                              ssem[kk2 % NBUF]).wait()
    plsc.subcore_barrier()
    pltpu.sync_copy(s_sh.at[pl.ds(s * ZROWS, ZROWS)],
                    out_hbm.at[c, pl.ds(s * ZROWS, ZROWS)])


RB = 2000


def _tc0_body(x_ref, w_ref, pre_ref):
    pre_ref[...] = jnp.dot(x_ref[...], w_ref[...],
                           preferred_element_type=jnp.float32)


def _tc1_body(deg_ref, pre_ref, qlo_ref, qhi_ref, dis_ref):
    deg = deg_ref[0] + deg_ref[1]
    dis = lax.rsqrt(deg + 1.0)
    q = pre_ref[...] * dis[:, 0:1]
    qlo_ref[...] = q[:, 0:CW]
    qhi_ref[...] = q[:, CW:D]
    dis_ref[...] = dis


def _tc2_body(slo_ref, shi_ref, qlo_ref, qhi_ref, dis_ref, w_ref,
              q2lo_ref, q2hi_ref):
    dis = dis_ref[:, 0:1]
    hlo = jnp.maximum((slo_ref[0] + slo_ref[1] + qlo_ref[...]) * dis, 0.0)
    hhi = jnp.maximum((shi_ref[0] + shi_ref[1] + qhi_ref[...]) * dis, 0.0)
    pre = (jnp.dot(hlo, w_ref[0:CW, :], preferred_element_type=jnp.float32)
           + jnp.dot(hhi, w_ref[CW:D, :], preferred_element_type=jnp.float32))
    q2 = pre * dis
    q2lo_ref[...] = q2[:, 0:CW]
    q2hi_ref[...] = q2[:, CW:D]


def _tc3_body(slo_ref, shi_ref, qlo_ref, qhi_ref, dis_ref, out_ref):
    dis = dis_ref[:, 0:1]
    out_ref[:, 0:CW] = jnp.maximum(
        (slo_ref[0] + slo_ref[1] + qlo_ref[...]) * dis, 0.0)
    out_ref[:, CW:D] = jnp.maximum(
        (shi_ref[0] + shi_ref[1] + qhi_ref[...]) * dis, 0.0)


_spec_nc = pl.BlockSpec((NC, RB, CW), lambda i: (0, i, 0))
_spec_deg = pl.BlockSpec((NC, RB, 16), lambda i: (0, i, 0))
_spec_q = pl.BlockSpec((RB, CW), lambda i: (i, 0))
_spec_dis = pl.BlockSpec((RB, 16), lambda i: (i, 0))
_spec_row = pl.BlockSpec((RB, D), lambda i: (i, 0))
_spec_w = pl.BlockSpec((D, D), lambda i: (0, 0))

_tc0 = pl.pallas_call(
    _tc0_body,
    grid=(N // RB,),
    in_specs=[_spec_row, _spec_w],
    out_specs=_spec_row,
    out_shape=jax.ShapeDtypeStruct((N, D), jnp.float32),
)

_tc1 = pl.pallas_call(
    _tc1_body,
    grid=(N // RB,),
    in_specs=[_spec_deg, _spec_row],
    out_specs=[_spec_q, _spec_q, _spec_dis],
    out_shape=[
        jax.ShapeDtypeStruct((N, CW), jnp.float32),
        jax.ShapeDtypeStruct((N, CW), jnp.float32),
        jax.ShapeDtypeStruct((N, 16), jnp.float32),
    ],
)

_tc2 = pl.pallas_call(
    _tc2_body,
    grid=(N // RB,),
    in_specs=[_spec_nc, _spec_nc, _spec_q, _spec_q, _spec_dis, _spec_w],
    out_specs=[_spec_q, _spec_q],
    out_shape=[
        jax.ShapeDtypeStruct((N, CW), jnp.float32),
        jax.ShapeDtypeStruct((N, CW), jnp.float32),
    ],
)

_tc3 = pl.pallas_call(
    _tc3_body,
    grid=(N // RB,),
    in_specs=[_spec_nc, _spec_nc, _spec_q, _spec_q, _spec_dis],
    out_specs=_spec_row,
    out_shape=jax.ShapeDtypeStruct((N, D), jnp.float32),
)


def kernel(x, edge_index, W1, W2):
    pad = jnp.tile(jnp.array([[0], [NPAD - 1]], jnp.int32), (1, EPAD - E))
    edge4 = jnp.concatenate([edge_index, pad], axis=1).reshape(
        2, NW, CPT, CH)
    pre1 = _tc0(x, W1)
    deg2 = _sc_degree(edge4)
    q1lo, q1hi, dis = _tc1(deg2, pre1)
    s1lo = _sc_spmm(q1lo, edge4)
    s1hi = _sc_spmm(q1hi, edge4)
    q2lo, q2hi = _tc2(s1lo, s1hi, q1lo, q1hi, dis, W2)
    s2lo = _sc_spmm(q2lo, edge4)
    s2hi = _sc_spmm(q2hi, edge4)
    return _tc3(s2lo, s2hi, q2lo, q2hi, dis)
